# baseline (speedup 1.0000x reference)
import functools

import jax
import jax.numpy as jnp
from jax import lax
from jax.experimental import pallas as pl
from jax.experimental.pallas import tpu as pltpu
from jax.experimental.pallas import tpu_sc as plsc

_IGNORE = 255
_RATIO = 0.7

_N, _C, _H, _W = 8, 19, 512, 512
_NPIX = _N * _H * _W
_NH = _N // 2
_HPIX = _NPIX // 2

_NW = 32
_SHARD = _HPIX // _NW
_B1 = 65536
_B2 = 32768


def _nll_body(x_ref, t_ref, w_ref):
    x = x_ref[...]
    tgt = t_ref[...]
    m = jnp.max(x, axis=1)
    se = jnp.sum(jnp.exp(x - m[:, None]), axis=1)
    cls = lax.broadcasted_iota(jnp.int32, x.shape, 1)
    xt = jnp.sum(jnp.where(cls == tgt[:, None], x, 0.0), axis=1)
    nll = jnp.log(se) - (xt - m)
    valid = tgt != _IGNORE
    w_ref[...] = lax.bitcast_convert_type(
        jnp.where(valid, nll, -1.0), jnp.int32
    )


def _nll_call(predict, target, half):
    return pl.pallas_call(
        _nll_body,
        grid=(_NH,),
        in_specs=[
            pl.BlockSpec((1, _C, _H, _W), lambda b: (b + half * _NH, 0, 0, 0)),
            pl.BlockSpec((1, _H, _W), lambda b: (b + half * _NH, 0, 0)),
        ],
        out_specs=pl.BlockSpec((1, _H, _W), lambda b: (b, 0, 0)),
        out_shape=jax.ShapeDtypeStruct((_NH, _H, _W), jnp.int32),
    )(predict, target)


@functools.partial(
    pl.kernel,
    mesh=plsc.VectorSubcoreMesh(core_axis_name="c", subcore_axis_name="s"),
    out_type=jax.ShapeDtypeStruct((_NW, _B1), jnp.int32),
    scratch_types=[
        pltpu.VMEM((_SHARD,), jnp.int32),
        pltpu.VMEM((_B1,), jnp.int32),
    ],
    compiler_params=pltpu.CompilerParams(needs_layout_passes=False),
)
def _hist_pass_a(bits_hbm, out_hbm, buf, hist):
    wid = lax.axis_index("s") * 2 + lax.axis_index("c")

    @plsc.parallel_loop(0, _B1 // 16, unroll=8)
    def _zero(i):
        hist[pl.ds(i * 16, 16)] = jnp.zeros((16,), jnp.int32)

    ones = jnp.ones((16,), jnp.int32)
    pltpu.sync_copy(bits_hbm.at[pl.ds(wid * _SHARD, _SHARD)], buf)

    @plsc.parallel_loop(0, _SHARD // 16, unroll=4)
    def _scatter(i):
        bits = buf[pl.ds(i * 16, 16)]
        ok = bits >= 0
        b = jnp.where(ok, bits >> 15, 0)
        plsc.addupdate_scatter(hist, [b], ones, mask=ok)

    pltpu.sync_copy(hist, out_hbm.at[wid])


@functools.partial(
    pl.kernel,
    mesh=plsc.VectorSubcoreMesh(core_axis_name="c", subcore_axis_name="s"),
    out_type=jax.ShapeDtypeStruct((_NW, _B2), jnp.int32),
    scratch_types=[
        pltpu.VMEM((_SHARD,), jnp.int32),
        pltpu.VMEM((_B2,), jnp.int32),
        pltpu.VMEM((16,), jnp.int32),
    ],
    compiler_params=pltpu.CompilerParams(needs_layout_passes=False),
)
def _hist_pass_b(bits0_hbm, bits1_hbm, b1_hbm, out_hbm, buf, hist, b1buf):
    wid = lax.axis_index("s") * 2 + lax.axis_index("c")
    pltpu.sync_copy(b1_hbm, b1buf)
    b1 = b1buf[...]

    @plsc.parallel_loop(0, _B2 // 16, unroll=8)
    def _zero(i):
        hist[pl.ds(i * 16, 16)] = jnp.zeros((16,), jnp.int32)

    ones = jnp.ones((16,), jnp.int32)

    for bits_hbm in (bits0_hbm, bits1_hbm):
        pltpu.sync_copy(bits_hbm.at[pl.ds(wid * _SHARD, _SHARD)], buf)

        @plsc.parallel_loop(0, _SHARD // 16, unroll=4)
        def _scatter(i):
            bits = buf[pl.ds(i * 16, 16)]
            ok = (bits >= 0) & ((bits >> 15) == b1)
            b = jnp.where(ok, bits & 0x7FFF, 0)
            plsc.addupdate_scatter(hist, [b], ones, mask=ok)

    pltpu.sync_copy(hist, out_hbm.at[wid])


def _search(h, binidx, k, nbins, iters):
    def lcount(m):
        return jnp.sum(jnp.where(binidx < m, h, 0))

    def body(_, lohi):
        lo, hi = lohi
        mid = (lo + hi) // 2
        above = lcount(mid + 1) >= k
        return (jnp.where(above, lo, mid), jnp.where(above, mid, hi))

    lo, hi = lax.fori_loop(
        0, iters, body, (jnp.int32(0), jnp.int32(nbins - 1))
    )
    return hi, lcount(hi)


def _scan_a_body(h0_ref, h1_ref, b1_ref, kp_ref):
    h = jnp.sum(h0_ref[...], axis=0) + jnp.sum(h1_ref[...], axis=0)
    r = lax.broadcasted_iota(jnp.int32, h.shape, 0)
    c = lax.broadcasted_iota(jnp.int32, h.shape, 1)
    binidx = r * 128 + c
    nv = jnp.sum(h)
    k = jnp.floor(nv.astype(jnp.float32) * _RATIO).astype(jnp.int32)
    b1, below = _search(h, binidx, k, _B1, 17)
    b1_ref[...] = jnp.full((1, 128), b1, jnp.int32)
    kp_ref[...] = jnp.full((1, 128), k - below, jnp.int32)


def _scan_a_call(h1a, h1b):
    return pl.pallas_call(
        _scan_a_body,
        out_shape=[
            jax.ShapeDtypeStruct((1, 128), jnp.int32),
            jax.ShapeDtypeStruct((1, 128), jnp.int32),
        ],
    )(h1a, h1b)


def _final_body(h_ref, b1_ref, kp_ref, w0_ref, w1_ref, out_ref):
    h = jnp.sum(h_ref[...], axis=0)
    r = lax.broadcasted_iota(jnp.int32, h.shape, 0)
    c = lax.broadcasted_iota(jnp.int32, h.shape, 1)
    binidx = r * 128 + c
    b2, _ = _search(h, binidx, kp_ref[0, 0], _B2, 16)
    t = (b1_ref[0, 0] << 15) | b2

    num = jnp.float32(0.0)
    den = jnp.float32(0.0)
    for w_ref in (w0_ref, w1_ref):
        w = w_ref[...]
        kept = w >= t
        num += jnp.sum(
            jnp.where(kept, lax.bitcast_convert_type(w, jnp.float32), 0.0)
        )
        den += jnp.sum(kept.astype(jnp.float32))
    out_ref[...] = jnp.full((1, 1), num / jnp.maximum(den, 1.0), jnp.float32)


def _final_call(h2, b1v, kpv, w0, w1):
    return pl.pallas_call(
        _final_body,
        in_specs=[
            pl.BlockSpec((_NW, _B2 // 128, 128), lambda: (0, 0, 0)),
            pl.BlockSpec(memory_space=pltpu.SMEM),
            pl.BlockSpec(memory_space=pltpu.SMEM),
            pl.BlockSpec((_NH, _H, _W), lambda: (0, 0, 0)),
            pl.BlockSpec((_NH, _H, _W), lambda: (0, 0, 0)),
        ],
        out_specs=pl.BlockSpec((1, 1), lambda: (0, 0)),
        out_shape=jax.ShapeDtypeStruct((1, 1), jnp.float32),
    )(h2, b1v, kpv, w0, w1)


def kernel(predict, target):
    w0 = _nll_call(predict, target, 0)
    w1 = _nll_call(predict, target, 1)
    wf0 = w0.reshape(_HPIX)
    wf1 = w1.reshape(_HPIX)
    h1a = _hist_pass_a(wf0)
    h1b = _hist_pass_a(wf1)
    b1v, kpv = _scan_a_call(
        h1a.reshape(_NW, _B1 // 128, 128), h1b.reshape(_NW, _B1 // 128, 128)
    )
    h2 = _hist_pass_b(wf0, wf1, b1v[0, :16])
    loss = _final_call(h2.reshape(_NW, _B2 // 128, 128), b1v, kpv, w0, w1)
    return loss[0, 0]


# speedup vs baseline: 1.0119x; 1.0119x over previous
import functools

import jax
import jax.numpy as jnp
from jax import lax
from jax.experimental import pallas as pl
from jax.experimental.pallas import tpu as pltpu
from jax.experimental.pallas import tpu_sc as plsc

_IGNORE = 255
_RATIO = 0.7

_N, _C, _H, _W = 8, 19, 512, 512
_NPIX = _N * _H * _W

_NW = 32
_PER_TILE = _NPIX // _NW
_SLAB = 32768
_B1 = 65536
_B2 = 32768
_S1 = _B1 // 16
_S2 = _B2 // 16


def _nll_body(x_ref, t_ref, w_ref):
    x = x_ref[...]
    tgt = t_ref[...]
    m = jnp.max(x, axis=1)
    se = jnp.sum(jnp.exp(x - m[:, None]), axis=1)
    cls = lax.broadcasted_iota(jnp.int32, x.shape, 1)
    xt = jnp.sum(jnp.where(cls == tgt[:, None], x, 0.0), axis=1)
    nll = jnp.log(se) - (xt - m)
    valid = tgt != _IGNORE
    w_ref[...] = lax.bitcast_convert_type(
        jnp.where(valid, nll, -1.0), jnp.int32
    )


def _nll_call(predict, target):
    return pl.pallas_call(
        _nll_body,
        grid=(_N,),
        in_specs=[
            pl.BlockSpec((1, _C, _H, _W), lambda b: (b, 0, 0, 0)),
            pl.BlockSpec((1, _H, _W), lambda b: (b, 0, 0)),
        ],
        out_specs=pl.BlockSpec((1, _H, _W), lambda b: (b, 0, 0)),
        out_shape=jax.ShapeDtypeStruct((_N, _H, _W), jnp.int32),
    )(predict, target)


def _merge_core(part_hbm, merged_hbm, tmp, acc, nbins, slice_len):
    c = lax.axis_index("c")
    s = lax.axis_index("s")
    off = s * slice_len

    plsc.subcore_barrier()

    @plsc.parallel_loop(0, slice_len // 16, unroll=8)
    def _zacc(i):
        acc[pl.ds(i * 16, 16)] = jnp.zeros((16,), jnp.int32)

    for w in range(16):
        pltpu.sync_copy(part_hbm.at[w * 2 + c, pl.ds(off, slice_len)], tmp)

        @plsc.parallel_loop(0, slice_len // 16, unroll=8)
        def _add(i):
            acc[pl.ds(i * 16, 16)] = (
                acc[pl.ds(i * 16, 16)] + tmp[pl.ds(i * 16, 16)]
            )

    pltpu.sync_copy(acc, merged_hbm.at[c, pl.ds(off, slice_len)])


@functools.partial(
    pl.kernel,
    mesh=plsc.VectorSubcoreMesh(core_axis_name="c", subcore_axis_name="s"),
    out_type=[
        jax.ShapeDtypeStruct((_NW, _B1), jnp.int32),
        jax.ShapeDtypeStruct((2, _B1), jnp.int32),
    ],
    scratch_types=[
        pltpu.VMEM((_SLAB,), jnp.int32),
        pltpu.VMEM((_B1,), jnp.int32),
        pltpu.VMEM((_S1,), jnp.int32),
        pltpu.VMEM((_S1,), jnp.int32),
    ],
    compiler_params=pltpu.CompilerParams(needs_layout_passes=False),
)
def _hist_pass_a(bits_hbm, part_hbm, merged_hbm, buf, hist, tmp, acc):
    wid = lax.axis_index("s") * 2 + lax.axis_index("c")
    base = wid * _PER_TILE

    @plsc.parallel_loop(0, _B1 // 16, unroll=8)
    def _zero(i):
        hist[pl.ds(i * 16, 16)] = jnp.zeros((16,), jnp.int32)

    ones = jnp.ones((16,), jnp.int32)

    def slab_body(s, c):
        pltpu.sync_copy(bits_hbm.at[pl.ds(base + s * _SLAB, _SLAB)], buf)

        @plsc.parallel_loop(0, _SLAB // 16, unroll=4)
        def _scatter(i):
            bits = buf[pl.ds(i * 16, 16)]
            ok = bits >= 0
            b = jnp.where(ok, bits >> 15, 0)
            plsc.addupdate_scatter(hist, [b], ones, mask=ok)

        return c

    lax.fori_loop(0, _PER_TILE // _SLAB, slab_body, 0)
    pltpu.sync_copy(hist, part_hbm.at[wid])
    _merge_core(part_hbm, merged_hbm, tmp, acc, _B1, _S1)


@functools.partial(
    pl.kernel,
    mesh=plsc.VectorSubcoreMesh(core_axis_name="c", subcore_axis_name="s"),
    out_type=[
        jax.ShapeDtypeStruct((_NW, _B2), jnp.int32),
        jax.ShapeDtypeStruct((2, _B2), jnp.int32),
    ],
    scratch_types=[
        pltpu.VMEM((_SLAB,), jnp.int32),
        pltpu.VMEM((_B2,), jnp.int32),
        pltpu.VMEM((16,), jnp.int32),
        pltpu.VMEM((_S2,), jnp.int32),
        pltpu.VMEM((_S2,), jnp.int32),
    ],
    compiler_params=pltpu.CompilerParams(needs_layout_passes=False),
)
def _hist_pass_b(bits_hbm, b1_hbm, part_hbm, merged_hbm,
                 buf, hist, b1buf, tmp, acc):
    wid = lax.axis_index("s") * 2 + lax.axis_index("c")
    base = wid * _PER_TILE
    pltpu.sync_copy(b1_hbm, b1buf)
    b1 = b1buf[...]

    @plsc.parallel_loop(0, _B2 // 16, unroll=8)
    def _zero(i):
        hist[pl.ds(i * 16, 16)] = jnp.zeros((16,), jnp.int32)

    ones = jnp.ones((16,), jnp.int32)

    def slab_body(s, c):
        pltpu.sync_copy(bits_hbm.at[pl.ds(base + s * _SLAB, _SLAB)], buf)

        @plsc.parallel_loop(0, _SLAB // 16, unroll=4)
        def _scatter(i):
            bits = buf[pl.ds(i * 16, 16)]
            ok = (bits >= 0) & ((bits >> 15) == b1)
            b = jnp.where(ok, bits & 0x7FFF, 0)
            plsc.addupdate_scatter(hist, [b], ones, mask=ok)

        return c

    lax.fori_loop(0, _PER_TILE // _SLAB, slab_body, 0)
    pltpu.sync_copy(hist, part_hbm.at[wid])
    _merge_core(part_hbm, merged_hbm, tmp, acc, _B2, _S2)


def _search(h, binidx, k, nbins, iters):
    def lcount(m):
        return jnp.sum(jnp.where(binidx < m, h, 0))

    def body(_, lohi):
        lo, hi = lohi
        mid = (lo + hi) // 2
        above = lcount(mid + 1) >= k
        return (jnp.where(above, lo, mid), jnp.where(above, mid, hi))

    lo, hi = lax.fori_loop(
        0, iters, body, (jnp.int32(0), jnp.int32(nbins - 1))
    )
    return hi, lcount(hi)


def _scan_a_body(h_ref, b1_ref, kp_ref):
    h = jnp.sum(h_ref[...], axis=0)
    r = lax.broadcasted_iota(jnp.int32, h.shape, 0)
    c = lax.broadcasted_iota(jnp.int32, h.shape, 1)
    binidx = r * 128 + c
    nv = jnp.sum(h)
    k = jnp.floor(nv.astype(jnp.float32) * _RATIO).astype(jnp.int32)
    b1, below = _search(h, binidx, k, _B1, 17)
    b1_ref[...] = jnp.full((1, 128), b1, jnp.int32)
    kp_ref[...] = jnp.full((1, 128), k - below, jnp.int32)


def _scan_a_call(h1):
    return pl.pallas_call(
        _scan_a_body,
        out_shape=[
            jax.ShapeDtypeStruct((1, 128), jnp.int32),
            jax.ShapeDtypeStruct((1, 128), jnp.int32),
        ],
    )(h1)


def _final_body(h_ref, b1_ref, kp_ref, w_ref, out_ref):
    h = jnp.sum(h_ref[...], axis=0)
    r = lax.broadcasted_iota(jnp.int32, h.shape, 0)
    c = lax.broadcasted_iota(jnp.int32, h.shape, 1)
    binidx = r * 128 + c
    b2, _ = _search(h, binidx, kp_ref[0, 0], _B2, 16)
    t = (b1_ref[0, 0] << 15) | b2

    w = w_ref[...]
    kept = w >= t
    num = jnp.sum(
        jnp.where(kept, lax.bitcast_convert_type(w, jnp.float32), 0.0)
    )
    den = jnp.sum(kept.astype(jnp.float32))
    out_ref[...] = jnp.full((1, 1), num / jnp.maximum(den, 1.0), jnp.float32)


def _final_call(h2, b1v, kpv, w):
    return pl.pallas_call(
        _final_body,
        in_specs=[
            pl.BlockSpec((2, _B2 // 128, 128), lambda: (0, 0, 0)),
            pl.BlockSpec(memory_space=pltpu.SMEM),
            pl.BlockSpec(memory_space=pltpu.SMEM),
            pl.BlockSpec((_N, _H, _W), lambda: (0, 0, 0)),
        ],
        out_specs=pl.BlockSpec((1, 1), lambda: (0, 0)),
        out_shape=jax.ShapeDtypeStruct((1, 1), jnp.float32),
    )(h2, b1v, kpv, w)


def kernel(predict, target):
    w = _nll_call(predict, target)
    wf = w.reshape(_NPIX)
    _, h1 = _hist_pass_a(wf)
    b1v, kpv = _scan_a_call(h1.reshape(2, _B1 // 128, 128))
    _, h2 = _hist_pass_b(wf, b1v[0, :16])
    loss = _final_call(h2.reshape(2, _B2 // 128, 128), b1v, kpv, w)
    return loss[0, 0]


# speedup vs baseline: 1.1342x; 1.1209x over previous
import functools

import jax
import jax.numpy as jnp
from jax import lax
from jax.experimental import pallas as pl
from jax.experimental.pallas import tpu as pltpu
from jax.experimental.pallas import tpu_sc as plsc

_IGNORE = 255
_RATIO = 0.7

_N, _C, _H, _W = 8, 19, 512, 512
_NPIX = _N * _H * _W

_NW = 32
_PER_TILE = _NPIX // _NW
_SLAB = 32768
_B1 = 65536
_B2 = 32768


def _nll_body(x_ref, t_ref, w_ref):
    x = x_ref[...]
    tgt = t_ref[...]
    m = jnp.max(x, axis=1)
    se = jnp.sum(jnp.exp(x - m[:, None]), axis=1)
    cls = lax.broadcasted_iota(jnp.int32, x.shape, 1)
    xt = jnp.sum(jnp.where(cls == tgt[:, None], x, 0.0), axis=1)
    nll = jnp.log(se) - (xt - m)
    valid = tgt != _IGNORE
    w_ref[...] = lax.bitcast_convert_type(
        jnp.where(valid, nll, -1.0), jnp.int32
    )


def _nll_call(predict, target):
    return pl.pallas_call(
        _nll_body,
        grid=(_N,),
        in_specs=[
            pl.BlockSpec((1, _C, _H, _W), lambda b: (b, 0, 0, 0)),
            pl.BlockSpec((1, _H, _W), lambda b: (b, 0, 0)),
        ],
        out_specs=pl.BlockSpec((1, _H, _W), lambda b: (b, 0, 0)),
        out_shape=jax.ShapeDtypeStruct((_N, _H, _W), jnp.int32),
    )(predict, target)


@functools.partial(
    pl.kernel,
    mesh=plsc.VectorSubcoreMesh(core_axis_name="c", subcore_axis_name="s"),
    out_type=jax.ShapeDtypeStruct((_NW, _B1), jnp.int32),
    scratch_types=[
        pltpu.VMEM((_SLAB,), jnp.int32),
        pltpu.VMEM((_B1,), jnp.int32),
    ],
    compiler_params=pltpu.CompilerParams(needs_layout_passes=False),
)
def _hist_pass_a(bits_hbm, part_hbm, buf, hist):
    wid = lax.axis_index("s") * 2 + lax.axis_index("c")
    base = wid * _PER_TILE

    @plsc.parallel_loop(0, _B1 // 16, unroll=8)
    def _zero(i):
        hist[pl.ds(i * 16, 16)] = jnp.zeros((16,), jnp.int32)

    ones = jnp.ones((16,), jnp.int32)

    def slab_body(s, c):
        pltpu.sync_copy(bits_hbm.at[pl.ds(base + s * _SLAB, _SLAB)], buf)

        @plsc.parallel_loop(0, _SLAB // 16, unroll=4)
        def _scatter(i):
            bits = buf[pl.ds(i * 16, 16)]
            ok = bits >= 0
            b = jnp.where(ok, bits >> 15, 0)
            plsc.addupdate_scatter(hist, [b], ones, mask=ok)

        return c

    lax.fori_loop(0, _PER_TILE // _SLAB, slab_body, 0)
    pltpu.sync_copy(hist, part_hbm.at[wid])


@functools.partial(
    pl.kernel,
    mesh=plsc.VectorSubcoreMesh(core_axis_name="c", subcore_axis_name="s"),
    out_type=jax.ShapeDtypeStruct((_NW, _B2), jnp.int32),
    scratch_types=[
        pltpu.VMEM((_SLAB,), jnp.int32),
        pltpu.VMEM((_B2,), jnp.int32),
        pltpu.VMEM((16,), jnp.int32),
    ],
    compiler_params=pltpu.CompilerParams(needs_layout_passes=False),
)
def _hist_pass_b(bits_hbm, b1_hbm, part_hbm, buf, hist, b1buf):
    wid = lax.axis_index("s") * 2 + lax.axis_index("c")
    base = wid * _PER_TILE
    pltpu.sync_copy(b1_hbm, b1buf)
    b1 = b1buf[...]

    @plsc.parallel_loop(0, _B2 // 16, unroll=8)
    def _zero(i):
        hist[pl.ds(i * 16, 16)] = jnp.zeros((16,), jnp.int32)

    ones = jnp.ones((16,), jnp.int32)

    def slab_body(s, c):
        pltpu.sync_copy(bits_hbm.at[pl.ds(base + s * _SLAB, _SLAB)], buf)

        @plsc.parallel_loop(0, _SLAB // 16, unroll=4)
        def _scatter(i):
            bits = buf[pl.ds(i * 16, 16)]
            ok = (bits >= 0) & ((bits >> 15) == b1)
            b = jnp.where(ok, bits & 0x7FFF, 0)
            plsc.addupdate_scatter(hist, [b], ones, mask=ok)

        return c

    lax.fori_loop(0, _PER_TILE // _SLAB, slab_body, 0)
    pltpu.sync_copy(hist, part_hbm.at[wid])


def _search(h, binidx, k, nbins, iters):
    def lcount(m):
        return jnp.sum(jnp.where(binidx < m, h, 0))

    def body(_, lohi):
        lo, hi = lohi
        mid = (lo + hi) // 2
        above = lcount(mid + 1) >= k
        return (jnp.where(above, lo, mid), jnp.where(above, mid, hi))

    lo, hi = lax.fori_loop(
        0, iters, body, (jnp.int32(0), jnp.int32(nbins - 1))
    )
    return hi, lcount(hi)


def _scan_a_body(h_ref, b1_ref, kp_ref):
    h = jnp.sum(h_ref[...], axis=0)
    r = lax.broadcasted_iota(jnp.int32, h.shape, 0)
    c = lax.broadcasted_iota(jnp.int32, h.shape, 1)
    binidx = r * 128 + c
    nv = jnp.sum(h)
    k = jnp.floor(nv.astype(jnp.float32) * _RATIO).astype(jnp.int32)
    b1, below = _search(h, binidx, k, _B1, 17)
    b1_ref[...] = jnp.full((1, 128), b1, jnp.int32)
    kp_ref[...] = jnp.full((1, 128), k - below, jnp.int32)


def _scan_a_call(h1):
    return pl.pallas_call(
        _scan_a_body,
        out_shape=[
            jax.ShapeDtypeStruct((1, 128), jnp.int32),
            jax.ShapeDtypeStruct((1, 128), jnp.int32),
        ],
    )(h1)


def _final_body(h_ref, b1_ref, kp_ref, w_ref, out_ref):
    h = jnp.sum(h_ref[...], axis=0)
    r = lax.broadcasted_iota(jnp.int32, h.shape, 0)
    c = lax.broadcasted_iota(jnp.int32, h.shape, 1)
    binidx = r * 128 + c
    b2, _ = _search(h, binidx, kp_ref[0, 0], _B2, 16)
    t = (b1_ref[0, 0] << 15) | b2

    w = w_ref[...]
    kept = w >= t
    num = jnp.sum(
        jnp.where(kept, lax.bitcast_convert_type(w, jnp.float32), 0.0)
    )
    den = jnp.sum(kept.astype(jnp.float32))
    out_ref[...] = jnp.full((1, 1), num / jnp.maximum(den, 1.0), jnp.float32)


def _final_call(h2, b1v, kpv, w):
    return pl.pallas_call(
        _final_body,
        in_specs=[
            pl.BlockSpec((_NW, _B2 // 128, 128), lambda: (0, 0, 0)),
            pl.BlockSpec(memory_space=pltpu.SMEM),
            pl.BlockSpec(memory_space=pltpu.SMEM),
            pl.BlockSpec((_N, _H, _W), lambda: (0, 0, 0)),
        ],
        out_specs=pl.BlockSpec((1, 1), lambda: (0, 0)),
        out_shape=jax.ShapeDtypeStruct((1, 1), jnp.float32),
    )(h2, b1v, kpv, w)


def kernel(predict, target):
    w = _nll_call(predict, target)
    wf = w.reshape(_NPIX)
    h1 = _hist_pass_a(wf)
    b1v, kpv = _scan_a_call(h1.reshape(_NW, _B1 // 128, 128))
    h2 = _hist_pass_b(wf, b1v[0, :16])
    loss = _final_call(h2.reshape(_NW, _B2 // 128, 128), b1v, kpv, w)
    return loss[0, 0]
